# baseline (speedup 1.0000x reference)
import functools

import jax
import jax.numpy as jnp
from jax import lax
from jax.experimental import pallas as pl
from jax.experimental.pallas import tpu as pltpu
from jax.experimental.pallas import tpu_sc as plsc

R = 64
N = 1_000_000
L = 16
NC = 2
NS = 16
NW = NC * NS
ROWS_PER_W = R // NW
CHUNK = 20000
NCHUNK = N // CHUNK
UNROLL = 10
INNER = CHUNK // L // UNROLL
BIG_I32 = 2147483647

_mesh = plsc.VectorSubcoreMesh(core_axis_name="c", subcore_axis_name="s")


@functools.partial(
    pl.kernel,
    out_type=(jax.ShapeDtypeStruct((NW, ROWS_PER_W, L), jnp.float32),
              jax.ShapeDtypeStruct((NW, ROWS_PER_W, L), jnp.int32)),
    mesh=_mesh,
    scratch_types=[
        pltpu.VMEM((CHUNK,), jnp.float32),
        pltpu.VMEM((CHUNK,), jnp.float32),
        pltpu.VMEM((CHUNK,), jnp.float32),
        pltpu.VMEM((CHUNK,), jnp.float32),
        pltpu.VMEM((ROWS_PER_W, L), jnp.float32),
        pltpu.VMEM((ROWS_PER_W, L), jnp.int32),
        pltpu.SemaphoreType.DMA,
        pltpu.SemaphoreType.DMA,
        pltpu.SemaphoreType.DMA,
        pltpu.SemaphoreType.DMA,
    ],
)
def _sc_argmax(scores_hbm, gumbel_hbm, outm_hbm, outi_hbm,
               sbuf0, sbuf1, gbuf0, gbuf1, res_m, res_i,
               sem_s0, sem_s1, sem_g0, sem_g1):
    wid = lax.axis_index("s") * NC + lax.axis_index("c")
    lane = lax.iota(jnp.int32, L)

    slots = ((sbuf0, gbuf0, sem_s0, sem_g0), (sbuf1, gbuf1, sem_s1, sem_g1))

    for r in range(ROWS_PER_W):
        row = wid * ROWS_PER_W + r

        for b, (sb, gb, ss, gs) in enumerate(slots):
            pltpu.async_copy(scores_hbm.at[row * NCHUNK + b], sb, ss)
            pltpu.async_copy(gumbel_hbm.at[row * NCHUNK + b], gb, gs)

        m0 = jnp.full((L,), -jnp.inf, jnp.float32)
        mi0 = jnp.zeros((L,), jnp.int32)
        idxv0 = lane

        def chunk_pair(c2, carry, row=row):
            for b, (sb, gb, ss, gs) in enumerate(slots):
                c = c2 * 2 + b
                pltpu.make_async_copy(scores_hbm.at[row * NCHUNK], sb, ss).wait()
                pltpu.make_async_copy(gumbel_hbm.at[row * NCHUNK], gb, gs).wait()

                def step(i, car, sb=sb, gb=gb):
                    m, mi, idxv = car
                    base = pl.multiple_of(i * (UNROLL * L), UNROLL * L)
                    for u in range(UNROLL):
                        off = base + u * L
                        v = sb[pl.ds(off, L)] + gb[pl.ds(off, L)]
                        upd = v > m
                        m = jnp.where(upd, v, m)
                        mi = jnp.where(upd, idxv + u * L, mi)
                    return m, mi, idxv + UNROLL * L

                carry = lax.fori_loop(0, INNER, step, carry)

                @pl.when(c + 2 < NCHUNK)
                def _(sb=sb, gb=gb, ss=ss, gs=gs, c=c, row=row):
                    pltpu.async_copy(scores_hbm.at[row * NCHUNK + c + 2], sb, ss)
                    pltpu.async_copy(gumbel_hbm.at[row * NCHUNK + c + 2], gb, gs)
            return carry

        m, mi, _ = lax.fori_loop(0, NCHUNK // 2, chunk_pair, (m0, mi0, idxv0))

        res_m.at[r][...] = m
        res_i.at[r][...] = mi

    pltpu.sync_copy(res_m, outm_hbm.at[wid])
    pltpu.sync_copy(res_i, outi_hbm.at[wid])


def kernel(scores, gumbel):
    s2 = scores.reshape(R * NCHUNK, CHUNK)
    g2 = gumbel.reshape(R * NCHUNK, CHUNK)
    outm, outi = _sc_argmax(s2, g2)
    m = outm.reshape(R, L)
    mi = outi.reshape(R, L)
    gmax = jnp.max(m, axis=1, keepdims=True)
    gidx = jnp.min(jnp.where(m == gmax, mi, BIG_I32), axis=1)
    return gidx[:, None].astype(jnp.int32)
